# baseline (speedup 1.0000x reference)
import functools

import jax
import jax.numpy as jnp
from jax import lax
from jax.experimental import pallas as pl
from jax.experimental.pallas import tpu as pltpu
from jax.experimental.pallas import tpu_sc as plsc

N = 10000
NP = 10240
E = 320000
G = 100
GP = 128
HID = 64
EPW = 10112
CPW = 79
LASTC = 51
NPT = NP // 16

_mesh = plsc.VectorSubcoreMesh(core_axis_name="c", subcore_axis_name="s")
_sc_params = pltpu.CompilerParams(use_tc_tiling_on_sc=False,
                                  needs_layout_passes=False)


def _copy_edges(ei, row, base, blk, last):
    @pl.when(jnp.logical_not(last))
    def _():
        pltpu.sync_copy(ei.at[row].at[pl.ds(base, EPW)],
                        blk.at[pl.ds(0, EPW)])

    @pl.when(last)
    def _():
        pltpu.sync_copy(ei.at[row].at[pl.ds(base, LASTC * 128)],
                        blk.at[pl.ds(0, LASTC * 128)])


@functools.partial(
    pl.kernel,
    out_type=(
        jax.ShapeDtypeStruct((2, NP), jnp.float32),
        jax.ShapeDtypeStruct((GP,), jnp.float32),
    ),
    mesh=_mesh,
    compiler_params=_sc_params,
    scratch_types=[
        pltpu.VMEM((EPW,), jnp.int32),
        pltpu.VMEM((640,), jnp.int32),
        pltpu.VMEM((128,), jnp.float32),
        pltpu.VMEM((128,), jnp.float32),
        pltpu.VMEM_SHARED((NP,), jnp.float32),
        pltpu.VMEM_SHARED((GP,), jnp.float32),
    ],
)
def _sc_a(ei, batch_h, deg_out, bc_out, dstblk, bblk, ones_v, zeros_v,
          deg_s, bc_s):
    cid = lax.axis_index("c")
    sid = lax.axis_index("s")
    wid = cid * 16 + sid
    last = wid == 31
    nch = jnp.where(last, LASTC, CPW)
    for k in range(8):
        ones_v[pl.ds(k * 16, 16)] = jnp.full((16,), 1.0, jnp.float32)
        zeros_v[pl.ds(k * 16, 16)] = jnp.zeros((16,), jnp.float32)

    for k in range(5):
        pltpu.sync_copy(zeros_v, deg_s.at[pl.ds(sid * NPT + k * 128, 128)])

    @pl.when(jnp.logical_and(cid == 1, sid == 0))
    def _():
        pltpu.sync_copy(zeros_v, bc_s)

    plsc.subcore_barrier()

    _copy_edges(ei, 1, wid * EPW, dstblk, last)

    def body(j, carry):
        pltpu.sync_copy(ones_v, deg_s.at[dstblk.at[pl.ds(j * 128, 128)]],
                        add=True)
        return carry

    lax.fori_loop(0, nch, body, 0)

    @pl.when(jnp.logical_and(cid == 1, sid < 15))
    def _():
        pltpu.sync_copy(batch_h.at[pl.ds(sid * 640, 640)], bblk)
        for j in range(5):
            pltpu.sync_copy(ones_v, bc_s.at[bblk.at[pl.ds(j * 128, 128)]],
                            add=True)

    @pl.when(jnp.logical_and(cid == 1, sid == 15))
    def _():
        pltpu.sync_copy(batch_h.at[pl.ds(9600, 400)], bblk.at[pl.ds(0, 400)])
        for j in range(3):
            pltpu.sync_copy(ones_v, bc_s.at[bblk.at[pl.ds(j * 128, 128)]],
                            add=True)
        pltpu.sync_copy(ones_v.at[pl.ds(0, 16)],
                        bc_s.at[bblk.at[pl.ds(384, 16)]], add=True)

    plsc.subcore_barrier()

    pltpu.sync_copy(deg_s.at[pl.ds(sid * NPT, NPT)],
                    deg_out.at[cid].at[pl.ds(sid * NPT, NPT)])

    @pl.when(jnp.logical_and(cid == 1, sid == 0))
    def _():
        pltpu.sync_copy(bc_s, bc_out)


def _tc_b_body(x_ref, w1_ref, b1_ref, deg_ref, bc_ref,
               y_ref, t1_ref, dinv_ref, idx_ref):
    xw = jnp.dot(x_ref[...], w1_ref[...], preferred_element_type=jnp.float32)
    xw = jnp.concatenate(
        [xw, jnp.zeros((NP - N, HID), jnp.float32)], axis=0)
    deg = deg_ref[0] + deg_ref[1] + 1.0
    dinv = lax.rsqrt(deg)
    y_ref[...] = (xw * dinv).astype(jnp.bfloat16)
    t1_ref[...] = xw * (dinv * dinv) + b1_ref[...][None, :]
    dinv_ref[...] = dinv
    ri = lax.broadcasted_iota(jnp.int32, (GP, GP), 0)
    ci = lax.broadcasted_iota(jnp.int32, (GP, GP), 1)
    tri = (ri <= ci).astype(jnp.float32)
    cs = jnp.dot(bc_ref[...], tri, preferred_element_type=jnp.float32)
    idxf = cs - 1.0
    idxf = jnp.where(idxf < 0.0, idxf + float(N), idxf)
    idx_ref[...] = idxf.astype(jnp.int32)


_tc_b = pl.pallas_call(
    _tc_b_body,
    out_shape=(
        jax.ShapeDtypeStruct((NP, HID), jnp.bfloat16),
        jax.ShapeDtypeStruct((NP, HID), jnp.float32),
        jax.ShapeDtypeStruct((NP, 1), jnp.float32),
        jax.ShapeDtypeStruct((1, GP), jnp.int32),
    ),
)


@functools.partial(
    pl.kernel,
    out_type=jax.ShapeDtypeStruct((4, NP, HID), jnp.bfloat16),
    mesh=_mesh,
    compiler_params=_sc_params,
    scratch_types=[
        pltpu.VMEM((EPW,), jnp.int32),
        pltpu.VMEM((EPW,), jnp.int32),
        pltpu.VMEM((2, 128, HID), jnp.bfloat16),
        pltpu.VMEM((128, HID), jnp.bfloat16),
        pltpu.VMEM_SHARED((NP, HID), jnp.bfloat16),
        [pltpu.VMEM_SHARED((NP, HID), jnp.bfloat16)] * 2,
        [pltpu.SemaphoreType.DMA] * 2,
    ],
)
def _sc_c(y_hbm, ei, acc_out, srcblk, dstblk, rows, zrows, y_s, accs,
          sems):
    cid = lax.axis_index("c")
    sid = lax.axis_index("s")
    wid = cid * 16 + sid
    last = wid == 31
    nch = jnp.where(last, LASTC, CPW)

    def zbody(r, carry):
        for k in range(HID // 32):
            zrows[r, pl.ds(k * 32, 32)] = jnp.zeros((32,), jnp.bfloat16)
        return carry

    lax.fori_loop(0, 128, zbody, 0)
    for a in range(2):
        for k in range(5):
            pltpu.sync_copy(zrows,
                            accs[a].at[pl.ds(sid * NPT + k * 128, 128)])
    pltpu.sync_copy(y_hbm.at[pl.ds(sid * NPT, NPT)],
                    y_s.at[pl.ds(sid * NPT, NPT)])
    plsc.subcore_barrier()

    _copy_edges(ei, 0, wid * EPW, srcblk, last)
    _copy_edges(ei, 1, wid * EPW, dstblk, last)

    for b in range(2):
        pltpu.async_copy(y_s.at[srcblk.at[pl.ds(b * 128, 128)]], rows.at[b],
                         sems[b])

    def body(jj, carry):
        for b in range(2):
            j = jj * 2 + b
            pltpu.make_async_copy(y_s.at[srcblk.at[pl.ds(0, 128)]],
                                  rows.at[b], sems[b]).wait()
            pltpu.sync_copy(rows.at[b],
                            accs[b].at[dstblk.at[pl.ds(j * 128, 128)]],
                            add=True)

            @pl.when(j + 2 < nch)
            def _():
                pltpu.async_copy(
                    y_s.at[srcblk.at[pl.ds((j + 2) * 128, 128)]],
                    rows.at[b], sems[b])

        return carry

    lax.fori_loop(0, (nch - 1) // 2, body, 0, unroll=False)

    def tail(j, carry):
        pltpu.make_async_copy(y_s.at[srcblk.at[pl.ds(0, 128)]],
                              rows.at[0], sems[0]).wait()
        pltpu.sync_copy(rows.at[0],
                        accs[0].at[dstblk.at[pl.ds(j * 128, 128)]],
                        add=True)
        return carry

    lax.fori_loop(nch - 1, nch, tail, 0)
    plsc.subcore_barrier()
    for a in range(2):
        for k in range(5):
            pltpu.sync_copy(
                accs[a].at[pl.ds(sid * NPT + k * 128, 128)],
                acc_out.at[cid * 2 + a].at[pl.ds(sid * NPT + k * 128, 128)])


def _tc_d_body(acc_ref, t1_ref, dinv_ref, w2_ref, b2_ref, v_ref, t2_ref):
    dinv = dinv_ref[...]
    agg = (acc_ref[0].astype(jnp.float32) + acc_ref[1].astype(jnp.float32)
           + acc_ref[2].astype(jnp.float32) + acc_ref[3].astype(jnp.float32))
    h = jnp.maximum(agg * dinv + t1_ref[...], 0.0)
    u = jnp.dot(h, w2_ref[...], preferred_element_type=jnp.float32)
    v_ref[...] = dinv * u
    t2_ref[...] = dinv * dinv * u + b2_ref[...][None, :]


_tc_d = pl.pallas_call(
    _tc_d_body,
    out_shape=(
        jax.ShapeDtypeStruct((NP, 1), jnp.float32),
        jax.ShapeDtypeStruct((NP, 1), jnp.float32),
    ),
)


@functools.partial(
    pl.kernel,
    out_type=jax.ShapeDtypeStruct((2, GP), jnp.float32),
    mesh=_mesh,
    compiler_params=_sc_params,
    scratch_types=[
        pltpu.VMEM((NP,), jnp.float32),
        pltpu.VMEM((EPW,), jnp.int32),
        pltpu.VMEM((EPW,), jnp.int32),
        pltpu.VMEM((128,), jnp.float32),
        pltpu.VMEM((128,), jnp.float32),
        pltpu.VMEM((NP,), jnp.float32),
        pltpu.VMEM((NP,), jnp.float32),
        pltpu.VMEM((NP,), jnp.float32),
        pltpu.VMEM((8, 16), jnp.int32),
        pltpu.VMEM((GP,), jnp.float32),
        pltpu.VMEM_SHARED((NP,), jnp.float32),
    ],
)
def _sc_e(v_hbm, ei, dinv_hbm, t2_hbm, idx_hbm, out_hbm,
          vloc, srcblk, dstblk, vals, zeros_v, qloc, dloc, tloc, iloc,
          oloc, q_s):
    cid = lax.axis_index("c")
    sid = lax.axis_index("s")
    wid = cid * 16 + sid
    last = wid == 31
    nch = jnp.where(last, LASTC, CPW)

    for k in range(8):
        zeros_v[pl.ds(k * 16, 16)] = jnp.zeros((16,), jnp.float32)
    for k in range(5):
        pltpu.sync_copy(zeros_v, q_s.at[pl.ds(sid * NPT + k * 128, 128)])
    plsc.subcore_barrier()

    pltpu.sync_copy(v_hbm, vloc)
    _copy_edges(ei, 0, wid * EPW, srcblk, last)
    _copy_edges(ei, 1, wid * EPW, dstblk, last)
    zi = jnp.zeros((16,), jnp.int32)

    def body(j, carry):
        for k in range(8):
            iv = srcblk[pl.ds(j * 128 + k * 16, 16)]
            vv = plsc.load_gather(vloc, [iv])
            vals[pl.ds(k * 16, 16)] = vv
        pltpu.sync_copy(vals, q_s.at[dstblk.at[pl.ds(j * 128, 128)]],
                        add=True)
        return carry

    lax.fori_loop(0, nch, body, 0)
    plsc.subcore_barrier()

    @pl.when(sid == 0)
    def _():
        pltpu.sync_copy(q_s, qloc)
        pltpu.sync_copy(dinv_hbm, dloc)
        pltpu.sync_copy(idx_hbm, iloc)

        @pl.when(cid == 0)
        def _():
            pltpu.sync_copy(t2_hbm, tloc)

        for k in range(8):
            ii = iloc[k]
            qv = plsc.load_gather(qloc, [ii])
            dv = plsc.load_gather(dloc, [ii])
            oloc[pl.ds(k * 16, 16)] = dv * qv

        @pl.when(cid == 0)
        def _():
            for k in range(8):
                ii = iloc[k]
                tv = plsc.load_gather(tloc, [ii])
                oloc[pl.ds(k * 16, 16)] = oloc[pl.ds(k * 16, 16)] + tv
        pltpu.sync_copy(oloc, out_hbm.at[cid])


def kernel(x, edge_index, batch, W1, b1, W2, b2):
    ei = edge_index.astype(jnp.int32)
    batch_h = batch.astype(jnp.int32)

    degp, bc = _sc_a(ei, batch_h)
    y, t1, dinv, idx = _tc_b(x, W1, b1, degp.reshape(2, NP, 1),
                             bc.reshape(1, GP))
    acc = _sc_c(y, ei)
    v, t2 = _tc_d(acc, t1, dinv, W2, b2)
    outp = _sc_e(v.reshape(NP), ei, dinv.reshape(NP), t2.reshape(NP),
                 idx.reshape(8, 16))
    return (outp[0] + outp[1])[:G]


# speedup vs baseline: 1.1428x; 1.1428x over previous
import functools

import jax
import jax.numpy as jnp
from jax import lax
from jax.experimental import pallas as pl
from jax.experimental.pallas import tpu as pltpu
from jax.experimental.pallas import tpu_sc as plsc

N = 10000
NP = 10240
E = 320000
G = 100
GP = 128
HID = 64
EPW = 10112
CPW = 79
LASTC = 51
NPT = NP // 16

_mesh = plsc.VectorSubcoreMesh(core_axis_name="c", subcore_axis_name="s")
_sc_params = pltpu.CompilerParams(use_tc_tiling_on_sc=False,
                                  needs_layout_passes=False)


def _copy_edges(ei, row, base, blk, last):
    @pl.when(jnp.logical_not(last))
    def _():
        pltpu.sync_copy(ei.at[row].at[pl.ds(base, EPW)],
                        blk.at[pl.ds(0, EPW)])

    @pl.when(last)
    def _():
        pltpu.sync_copy(ei.at[row].at[pl.ds(base, LASTC * 128)],
                        blk.at[pl.ds(0, LASTC * 128)])


@functools.partial(
    pl.kernel,
    out_type=(
        jax.ShapeDtypeStruct((2, NP), jnp.float32),
        jax.ShapeDtypeStruct((GP,), jnp.float32),
    ),
    mesh=_mesh,
    compiler_params=_sc_params,
    scratch_types=[
        pltpu.VMEM((EPW,), jnp.int32),
        pltpu.VMEM((640,), jnp.int32),
        pltpu.VMEM((128,), jnp.float32),
        pltpu.VMEM((128,), jnp.float32),
        pltpu.VMEM_SHARED((NP,), jnp.float32),
        pltpu.VMEM_SHARED((GP,), jnp.float32),
    ],
)
def _sc_a(ei, batch_h, deg_out, bc_out, dstblk, bblk, ones_v, zeros_v,
          deg_s, bc_s):
    cid = lax.axis_index("c")
    sid = lax.axis_index("s")
    wid = cid * 16 + sid
    last = wid == 31
    nch = jnp.where(last, LASTC, CPW)
    for k in range(8):
        ones_v[pl.ds(k * 16, 16)] = jnp.full((16,), 1.0, jnp.float32)
        zeros_v[pl.ds(k * 16, 16)] = jnp.zeros((16,), jnp.float32)

    for k in range(5):
        pltpu.sync_copy(zeros_v, deg_s.at[pl.ds(sid * NPT + k * 128, 128)])

    @pl.when(jnp.logical_and(cid == 1, sid == 0))
    def _():
        pltpu.sync_copy(zeros_v, bc_s)

    plsc.subcore_barrier()

    _copy_edges(ei, 1, wid * EPW, dstblk, last)

    def body(j, carry):
        pltpu.sync_copy(ones_v, deg_s.at[dstblk.at[pl.ds(j * 128, 128)]],
                        add=True)
        return carry

    lax.fori_loop(0, nch, body, 0)

    @pl.when(jnp.logical_and(cid == 1, sid < 15))
    def _():
        pltpu.sync_copy(batch_h.at[pl.ds(sid * 640, 640)], bblk)
        for j in range(5):
            pltpu.sync_copy(ones_v, bc_s.at[bblk.at[pl.ds(j * 128, 128)]],
                            add=True)

    @pl.when(jnp.logical_and(cid == 1, sid == 15))
    def _():
        pltpu.sync_copy(batch_h.at[pl.ds(9600, 400)], bblk.at[pl.ds(0, 400)])
        for j in range(3):
            pltpu.sync_copy(ones_v, bc_s.at[bblk.at[pl.ds(j * 128, 128)]],
                            add=True)
        pltpu.sync_copy(ones_v.at[pl.ds(0, 16)],
                        bc_s.at[bblk.at[pl.ds(384, 16)]], add=True)

    plsc.subcore_barrier()

    pltpu.sync_copy(deg_s.at[pl.ds(sid * NPT, NPT)],
                    deg_out.at[cid].at[pl.ds(sid * NPT, NPT)])

    @pl.when(jnp.logical_and(cid == 1, sid == 0))
    def _():
        pltpu.sync_copy(bc_s, bc_out)


def _tc_b_body(x_ref, w1_ref, b1_ref, deg_ref, bc_ref,
               y_ref, t1_ref, dinv_ref, idx_ref):
    xw = jnp.dot(x_ref[...], w1_ref[...], preferred_element_type=jnp.float32)
    xw = jnp.concatenate(
        [xw, jnp.zeros((NP - N, HID), jnp.float32)], axis=0)
    deg = deg_ref[0] + deg_ref[1] + 1.0
    dinv = lax.rsqrt(deg)
    y_ref[...] = (xw * dinv).astype(jnp.bfloat16)
    t1_ref[...] = xw * (dinv * dinv) + b1_ref[...][None, :]
    dinv_ref[...] = dinv
    ri = lax.broadcasted_iota(jnp.int32, (GP, GP), 0)
    ci = lax.broadcasted_iota(jnp.int32, (GP, GP), 1)
    tri = (ri <= ci).astype(jnp.float32)
    cs = jnp.dot(bc_ref[...], tri, preferred_element_type=jnp.float32)
    idxf = cs - 1.0
    idxf = jnp.where(idxf < 0.0, idxf + float(N), idxf)
    idx_ref[...] = idxf.astype(jnp.int32)


_tc_b = pl.pallas_call(
    _tc_b_body,
    out_shape=(
        jax.ShapeDtypeStruct((NP, HID), jnp.bfloat16),
        jax.ShapeDtypeStruct((NP, HID), jnp.float32),
        jax.ShapeDtypeStruct((NP, 1), jnp.float32),
        jax.ShapeDtypeStruct((1, GP), jnp.int32),
    ),
)


@functools.partial(
    pl.kernel,
    out_type=jax.ShapeDtypeStruct((2, NP, HID), jnp.bfloat16),
    mesh=_mesh,
    compiler_params=_sc_params,
    scratch_types=[
        pltpu.VMEM((EPW,), jnp.int32),
        pltpu.VMEM((EPW,), jnp.int32),
        pltpu.VMEM((2, 128, HID), jnp.bfloat16),
        pltpu.VMEM((128, HID), jnp.bfloat16),
        pltpu.VMEM_SHARED((NP, HID), jnp.bfloat16),
        pltpu.VMEM_SHARED((NP, HID), jnp.bfloat16),
        [pltpu.SemaphoreType.DMA] * 2,
    ],
)
def _sc_c(y_hbm, ei, acc_out, srcblk, dstblk, rows, zrows, y_s, acc_s,
          sems):
    cid = lax.axis_index("c")
    sid = lax.axis_index("s")
    wid = cid * 16 + sid
    last = wid == 31
    nch = jnp.where(last, LASTC, CPW)

    def zbody(r, carry):
        for k in range(HID // 32):
            zrows[r, pl.ds(k * 32, 32)] = jnp.zeros((32,), jnp.bfloat16)
        return carry

    lax.fori_loop(0, 128, zbody, 0)
    for k in range(5):
        pltpu.sync_copy(zrows, acc_s.at[pl.ds(sid * NPT + k * 128, 128)])
    pltpu.sync_copy(y_hbm.at[pl.ds(sid * NPT, NPT)],
                    y_s.at[pl.ds(sid * NPT, NPT)])
    plsc.subcore_barrier()

    _copy_edges(ei, 0, wid * EPW, srcblk, last)
    _copy_edges(ei, 1, wid * EPW, dstblk, last)

    for b in range(2):
        pltpu.async_copy(y_s.at[srcblk.at[pl.ds(b * 128, 128)]], rows.at[b],
                         sems[b])

    def body(jj, carry):
        for b in range(2):
            j = jj * 2 + b
            pltpu.make_async_copy(y_s.at[srcblk.at[pl.ds(0, 128)]],
                                  rows.at[b], sems[b]).wait()
            pltpu.sync_copy(rows.at[b],
                            acc_s.at[dstblk.at[pl.ds(j * 128, 128)]],
                            add=True)

            @pl.when(j + 2 < nch)
            def _():
                pltpu.async_copy(
                    y_s.at[srcblk.at[pl.ds((j + 2) * 128, 128)]],
                    rows.at[b], sems[b])

        return carry

    lax.fori_loop(0, (nch - 1) // 2, body, 0, unroll=False)

    def tail(j, carry):
        pltpu.make_async_copy(y_s.at[srcblk.at[pl.ds(0, 128)]],
                              rows.at[0], sems[0]).wait()
        pltpu.sync_copy(rows.at[0],
                        acc_s.at[dstblk.at[pl.ds(j * 128, 128)]],
                        add=True)
        return carry

    lax.fori_loop(nch - 1, nch, tail, 0)
    plsc.subcore_barrier()
    for k in range(5):
        pltpu.sync_copy(acc_s.at[pl.ds(sid * NPT + k * 128, 128)],
                        acc_out.at[cid].at[pl.ds(sid * NPT + k * 128, 128)])


def _tc_d_body(acc_ref, t1_ref, dinv_ref, w2_ref, b2_ref, v_ref, t2_ref):
    dinv = dinv_ref[...]
    agg = acc_ref[0].astype(jnp.float32) + acc_ref[1].astype(jnp.float32)
    h = jnp.maximum(agg * dinv + t1_ref[...], 0.0)
    u = jnp.dot(h, w2_ref[...], preferred_element_type=jnp.float32)
    v_ref[...] = dinv * u
    t2_ref[...] = dinv * dinv * u + b2_ref[...][None, :]


_tc_d = pl.pallas_call(
    _tc_d_body,
    out_shape=(
        jax.ShapeDtypeStruct((NP, 1), jnp.float32),
        jax.ShapeDtypeStruct((NP, 1), jnp.float32),
    ),
)


@functools.partial(
    pl.kernel,
    out_type=jax.ShapeDtypeStruct((2, GP), jnp.float32),
    mesh=_mesh,
    compiler_params=_sc_params,
    scratch_types=[
        pltpu.VMEM((NP,), jnp.float32),
        pltpu.VMEM((EPW,), jnp.int32),
        pltpu.VMEM((EPW,), jnp.int32),
        pltpu.VMEM((128,), jnp.float32),
        pltpu.VMEM((128,), jnp.float32),
        pltpu.VMEM((NP,), jnp.float32),
        pltpu.VMEM((NP,), jnp.float32),
        pltpu.VMEM((NP,), jnp.float32),
        pltpu.VMEM((8, 16), jnp.int32),
        pltpu.VMEM((GP,), jnp.float32),
        pltpu.VMEM_SHARED((NP,), jnp.float32),
    ],
)
def _sc_e(v_hbm, ei, dinv_hbm, t2_hbm, idx_hbm, out_hbm,
          vloc, srcblk, dstblk, vals, zeros_v, qloc, dloc, tloc, iloc,
          oloc, q_s):
    cid = lax.axis_index("c")
    sid = lax.axis_index("s")
    wid = cid * 16 + sid
    last = wid == 31
    nch = jnp.where(last, LASTC, CPW)

    for k in range(8):
        zeros_v[pl.ds(k * 16, 16)] = jnp.zeros((16,), jnp.float32)
    for k in range(5):
        pltpu.sync_copy(zeros_v, q_s.at[pl.ds(sid * NPT + k * 128, 128)])
    plsc.subcore_barrier()

    pltpu.sync_copy(v_hbm, vloc)
    _copy_edges(ei, 0, wid * EPW, srcblk, last)
    _copy_edges(ei, 1, wid * EPW, dstblk, last)
    zi = jnp.zeros((16,), jnp.int32)

    def body(j, carry):
        for k in range(8):
            iv = srcblk[pl.ds(j * 128 + k * 16, 16)]
            vv = plsc.load_gather(vloc, [iv])
            vals[pl.ds(k * 16, 16)] = vv
        pltpu.sync_copy(vals, q_s.at[dstblk.at[pl.ds(j * 128, 128)]],
                        add=True)
        return carry

    lax.fori_loop(0, nch, body, 0)
    plsc.subcore_barrier()

    @pl.when(sid == 0)
    def _():
        pltpu.sync_copy(q_s, qloc)
        pltpu.sync_copy(dinv_hbm, dloc)
        pltpu.sync_copy(idx_hbm, iloc)

        @pl.when(cid == 0)
        def _():
            pltpu.sync_copy(t2_hbm, tloc)

        for k in range(8):
            ii = iloc[k]
            qv = plsc.load_gather(qloc, [ii])
            dv = plsc.load_gather(dloc, [ii])
            oloc[pl.ds(k * 16, 16)] = dv * qv

        @pl.when(cid == 0)
        def _():
            for k in range(8):
                ii = iloc[k]
                tv = plsc.load_gather(tloc, [ii])
                oloc[pl.ds(k * 16, 16)] = oloc[pl.ds(k * 16, 16)] + tv
        pltpu.sync_copy(oloc, out_hbm.at[cid])


def kernel(x, edge_index, batch, W1, b1, W2, b2):
    ei = edge_index.astype(jnp.int32)
    batch_h = batch.astype(jnp.int32)

    degp, bc = _sc_a(ei, batch_h)
    y, t1, dinv, idx = _tc_b(x, W1, b1, degp.reshape(2, NP, 1),
                             bc.reshape(1, GP))
    acc = _sc_c(y, ei)
    v, t2 = _tc_d(acc, t1, dinv, W2, b2)
    outp = _sc_e(v.reshape(NP), ei, dinv.reshape(NP), t2.reshape(NP),
                 idx.reshape(8, 16))
    return (outp[0] + outp[1])[:G]
